# Initial kernel scaffold; baseline (speedup 1.0000x reference)
import jax
import jax.numpy as jnp
from jax.experimental import pallas as pl


def kernel(edge_index, node_atts, batch, emb, W_msg0, W_self0, b0, W_msg1, W_self1, b1, W_mean, b_mean, W_var, b_var, W_dec, emb_dec):
    raise NotImplementedError("write your pallas kernel here")


# speedup vs baseline: 11.6372x; 11.6372x over previous
import dataclasses
import functools

import jax
import jax.numpy as jnp
from jax import lax
from jax.experimental import pallas as pl
from jax.experimental.pallas import tpu as pltpu
from jax.experimental.pallas import tpu_sc as plsc

NDIM = 128
SDIM = 64
N_NODES = 10000
N_EDGES = 320000
N_GRAPHS = 256
N_ATTS = 32
BETA = 0.005

NC = 2
NS = 16
NW = NC * NS
EROWS = N_EDGES // 128
ROWS_PER_TILE = EROWS // NW
EXTRA_ROWS = EROWS - ROWS_PER_TILE * NW
NPAD = 10240

_HIGH = jax.lax.Precision.HIGHEST


def _mesh():
  return plsc.VectorSubcoreMesh(core_axis_name="c", subcore_axis_name="s")


def _sc_params():
  cp = pltpu.CompilerParams()
  if "needs_layout_passes" in pltpu.CompilerParams.__dataclass_fields__:
    cp = dataclasses.replace(cp, needs_layout_passes=False)
  return cp


def _sc_agg(h, ei3):
  @functools.partial(
      pl.kernel,
      out_type=jax.ShapeDtypeStruct((NC, N_NODES, NDIM), jnp.float32),
      mesh=_mesh(),
      scratch_types=[
          pltpu.VMEM((128,), jnp.int32),
          pltpu.VMEM((128,), jnp.int32),
          pltpu.VMEM((128, NDIM), jnp.float32),
          pltpu.VMEM((128,), jnp.int32),
          pltpu.VMEM((128,), jnp.int32),
          pltpu.VMEM((128, NDIM), jnp.float32),
          pltpu.VMEM_SHARED((N_NODES, NDIM), jnp.float32),
          pltpu.SemaphoreType.DMA,
          pltpu.SemaphoreType.DMA,
      ],
  )
  def k(h_hbm, ei_hbm, out_hbm, srcA, dstA, rowsA, srcB, dstB, rowsB,
        agg_sh, semA, semB):
    cid = lax.axis_index("c")
    sid = lax.axis_index("s")
    tid = cid * NS + sid

    z16 = jnp.zeros((16,), jnp.float32)

    @pl.loop(0, 128)
    def _(r):
      for c in range(NDIM // 16):
        rowsA[r, pl.ds(c * 16, 16)] = z16

    @pl.loop(0, 4)
    def _(b):
      pltpu.sync_copy(rowsA, agg_sh.at[pl.ds(sid * 624 + b * 128, 128)])

    pltpu.sync_copy(rowsA.at[pl.ds(0, 112)],
                    agg_sh.at[pl.ds(sid * 624 + 512, 112)])

    @pl.when(sid == 0)
    def _():
      pltpu.sync_copy(rowsA.at[pl.ds(0, 16)], agg_sh.at[pl.ds(9984, 16)])

    plsc.subcore_barrier()

    r0 = tid * ROWS_PER_TILE

    pltpu.sync_copy(ei_hbm.at[0, r0], srcA)
    pltpu.make_async_copy(h_hbm.at[srcA], rowsA, semA).start()

    @pl.loop(0, ROWS_PER_TILE // 2)
    def _(j):
      rA = r0 + 2 * j
      rB = rA + 1
      pltpu.sync_copy(ei_hbm.at[0, rB], srcB)
      pltpu.make_async_copy(h_hbm.at[srcB], rowsB, semB).start()
      pltpu.make_async_copy(h_hbm.at[srcA], rowsA, semA).wait()
      pltpu.sync_copy(ei_hbm.at[1, rA], dstA)
      pltpu.sync_copy(rowsA, agg_sh.at[dstA], add=True)
      @pl.when(j < ROWS_PER_TILE // 2 - 1)
      def _():
        pltpu.sync_copy(ei_hbm.at[0, rA + 2], srcA)
        pltpu.make_async_copy(h_hbm.at[srcA], rowsA, semA).start()
      pltpu.make_async_copy(h_hbm.at[srcB], rowsB, semB).wait()
      pltpu.sync_copy(ei_hbm.at[1, rB], dstB)
      pltpu.sync_copy(rowsB, agg_sh.at[dstB], add=True)

    @pl.when(tid < EXTRA_ROWS)
    def _():
      r = NW * ROWS_PER_TILE + tid
      pltpu.sync_copy(ei_hbm.at[0, r], srcA)
      pltpu.async_copy(h_hbm.at[srcA], rowsA, semA).wait()
      pltpu.sync_copy(ei_hbm.at[1, r], dstA)
      pltpu.sync_copy(rowsA, agg_sh.at[dstA], add=True)

    plsc.subcore_barrier()

    pltpu.sync_copy(agg_sh.at[pl.ds(sid * 624, 624)],
                    out_hbm.at[cid].at[pl.ds(sid * 624, 624)])

    @pl.when(sid == 0)
    def _():
      pltpu.sync_copy(agg_sh.at[pl.ds(9984, 16)],
                      out_hbm.at[cid].at[pl.ds(9984, 16)])

  return k(h, ei3)


_CH = 26


def _sc_decoder(ei3, batch_pad, atts_pad, qflat, aflat, lse):
  @functools.partial(
      pl.kernel,
      out_type=[
          jax.ShapeDtypeStruct((EROWS, 128), jnp.float32),
          jax.ShapeDtypeStruct((NW, 16), jnp.float32),
      ],
      mesh=_mesh(),
      compiler_params=_sc_params(),
      scratch_types=[
          pltpu.VMEM((NPAD,), jnp.int32),
          pltpu.VMEM((NPAD,), jnp.int32),
          pltpu.VMEM((N_GRAPHS * N_GRAPHS,), jnp.float32),
          pltpu.VMEM((N_GRAPHS * N_ATTS,), jnp.float32),
          pltpu.VMEM((N_GRAPHS,), jnp.float32),
          pltpu.VMEM((8, 128), jnp.int32),
          pltpu.VMEM((8, 128), jnp.int32),
          pltpu.VMEM((8, 128), jnp.float32),
          pltpu.VMEM((16,), jnp.float32),
      ],
  )
  def k(ei_hbm, b_hbm, a_hbm, q_hbm, af_hbm, lse_hbm, qout_hbm, att_hbm,
        b_v, a_v, Q_v, A_v, l_v, src_v, dst_v, qc_v, acc_v):
    cid = lax.axis_index("c")
    sid = lax.axis_index("s")
    tid = cid * NS + sid

    pltpu.sync_copy(b_hbm, b_v)
    pltpu.sync_copy(a_hbm, a_v)
    pltpu.sync_copy(q_hbm, Q_v)
    pltpu.sync_copy(af_hbm, A_v)
    pltpu.sync_copy(lse_hbm, l_v)

    def do_rows(row_base, nrows):
      row_base = pl.multiple_of(row_base, 8)
      pltpu.sync_copy(ei_hbm.at[0, pl.ds(row_base, nrows)],
                      src_v.at[pl.ds(0, nrows)])
      pltpu.sync_copy(ei_hbm.at[1, pl.ds(row_base, nrows)],
                      dst_v.at[pl.ds(0, nrows)])

      @pl.loop(0, nrows)
      def _(i):
        for c in range(8):
          s16 = src_v[i, pl.ds(c * 16, 16)]
          d16 = dst_v[i, pl.ds(c * 16, 16)]
          bs = plsc.load_gather(b_v, [s16])
          bd = plsc.load_gather(b_v, [d16])
          q16 = plsc.load_gather(Q_v, [bs * N_GRAPHS + bd])
          qc_v[i, pl.ds(c * 16, 16)] = q16

      pltpu.sync_copy(qc_v.at[pl.ds(0, nrows)],
                      qout_hbm.at[pl.ds(row_base, nrows)])

    nchunks = jnp.where(tid < 24, 10, 9)

    @pl.loop(0, nchunks)
    def _(j):
      do_rows(8 * (tid + NW * j), 8)

    @pl.when(tid == NW - 1)
    def _():
      do_rows(8 * 312, 4)

    nbase = tid * (NPAD // NW)

    def att_step(j, acc):
      n0 = pl.multiple_of(nbase + j * 16, 16)
      b16 = b_v[pl.ds(n0, 16)]
      a16 = a_v[pl.ds(n0, 16)]
      l16 = plsc.load_gather(l_v, [b16])
      af16 = plsc.load_gather(A_v, [b16 * N_ATTS + a16])
      nid = n0 + lax.iota(jnp.int32, 16)
      return acc + jnp.where(nid < N_NODES, l16 - af16, 0.0)

    acc = jnp.zeros((16,), jnp.float32)
    acc = pl.loop(0, NPAD // NW // 16, init_carry=acc)(att_step)
    acc_v[...] = acc
    pltpu.sync_copy(acc_v, att_hbm.at[tid])

  return k(ei3, batch_pad, atts_pad, qflat, aflat, lse)


_BLK = 1000


def _tc_embed(atts3, emb):
  def body(a_ref, e_ref, o_ref):
    a = a_ref[0, 0, :]
    onehot = (a[:, None] == lax.broadcasted_iota(jnp.int32, (_BLK, N_ATTS), 1)
              ).astype(jnp.float32)
    o_ref[...] = jnp.dot(onehot, e_ref[...], precision=_HIGH,
                         preferred_element_type=jnp.float32)

  return pl.pallas_call(
      body,
      grid=(N_NODES // _BLK,),
      in_specs=[
          pl.BlockSpec((1, 1, _BLK), lambda i: (i, 0, 0)),
          pl.BlockSpec((N_ATTS, NDIM), lambda i: (0, 0)),
      ],
      out_specs=pl.BlockSpec((_BLK, NDIM), lambda i: (i, 0)),
      out_shape=jax.ShapeDtypeStruct((N_NODES, NDIM), jnp.float32),
  )(atts3, emb)


def _tc_layer(p0, p1, h, Wm, Ws, b2):
  def body(p0_ref, p1_ref, h_ref, wm_ref, ws_ref, b_ref, o_ref):
    agg = p0_ref[...] + p1_ref[...]
    x = (jnp.dot(agg, wm_ref[...], precision=_HIGH,
                 preferred_element_type=jnp.float32)
         + jnp.dot(h_ref[...], ws_ref[...], precision=_HIGH,
                   preferred_element_type=jnp.float32)
         + b_ref[...])
    o_ref[...] = jnp.tanh(x)

  return pl.pallas_call(
      body,
      grid=(N_NODES // _BLK,),
      in_specs=[
          pl.BlockSpec((_BLK, NDIM), lambda i: (i, 0)),
          pl.BlockSpec((_BLK, NDIM), lambda i: (i, 0)),
          pl.BlockSpec((_BLK, NDIM), lambda i: (i, 0)),
          pl.BlockSpec((NDIM, NDIM), lambda i: (0, 0)),
          pl.BlockSpec((NDIM, NDIM), lambda i: (0, 0)),
          pl.BlockSpec((1, NDIM), lambda i: (0, 0)),
      ],
      out_specs=pl.BlockSpec((_BLK, NDIM), lambda i: (i, 0)),
      out_shape=jax.ShapeDtypeStruct((N_NODES, NDIM), jnp.float32),
  )(p0, p1, h, Wm, Ws, b2)


def _tc_layer_pool(p0, p1, h, Wm, Ws, b2, batch3):
  def body(p0_ref, p1_ref, h_ref, wm_ref, ws_ref, b_ref, bat_ref, g_ref):
    i = pl.program_id(0)
    agg = p0_ref[...] + p1_ref[...]
    x = (jnp.dot(agg, wm_ref[...], precision=_HIGH,
                 preferred_element_type=jnp.float32)
         + jnp.dot(h_ref[...], ws_ref[...], precision=_HIGH,
                   preferred_element_type=jnp.float32)
         + b_ref[...])
    h2 = jnp.tanh(x)
    bat = bat_ref[0, 0, :]
    onehot = (bat[:, None] == lax.broadcasted_iota(jnp.int32,
                                                   (_BLK, N_GRAPHS), 1)
              ).astype(jnp.float32)
    gpart = lax.dot_general(onehot, h2, (((0,), (0,)), ((), ())),
                            precision=_HIGH,
                            preferred_element_type=jnp.float32)

    @pl.when(i == 0)
    def _():
      g_ref[...] = jnp.zeros_like(g_ref)

    g_ref[...] += gpart

  return pl.pallas_call(
      body,
      grid=(N_NODES // _BLK,),
      in_specs=[
          pl.BlockSpec((_BLK, NDIM), lambda i: (i, 0)),
          pl.BlockSpec((_BLK, NDIM), lambda i: (i, 0)),
          pl.BlockSpec((_BLK, NDIM), lambda i: (i, 0)),
          pl.BlockSpec((NDIM, NDIM), lambda i: (0, 0)),
          pl.BlockSpec((NDIM, NDIM), lambda i: (0, 0)),
          pl.BlockSpec((1, NDIM), lambda i: (0, 0)),
          pl.BlockSpec((1, 1, _BLK), lambda i: (i, 0, 0)),
      ],
      out_specs=pl.BlockSpec((N_GRAPHS, NDIM), lambda i: (0, 0)),
      out_shape=jax.ShapeDtypeStruct((N_GRAPHS, NDIM), jnp.float32),
  )(p0, p1, h, Wm, Ws, b2, batch3)


def _tc_heads(g, W_mean, b_mean, W_var, b_var, W_dec, emb_dec):
  def body(g_ref, wm_ref, bm_ref, wv_ref, bv_ref, wd_ref, ed_ref,
           q_ref, a_ref, lse_ref, kl_ref):
    gg = g_ref[...]
    mean = jnp.dot(gg, wm_ref[...], precision=_HIGH,
                   preferred_element_type=jnp.float32) + bm_ref[...]
    var = jnp.dot(gg, wv_ref[...], precision=_HIGH,
                  preferred_element_type=jnp.float32) + bv_ref[...]
    kl = -0.5 * jnp.sum(1.0 + var - jnp.square(mean) - jnp.exp(var))
    zc = jnp.dot(mean, wd_ref[...], precision=_HIGH,
                 preferred_element_type=jnp.float32)
    q_ref[...] = lax.dot_general(zc, zc, (((1,), (1,)), ((), ())),
                                 precision=_HIGH,
                                 preferred_element_type=jnp.float32)
    A = jnp.dot(zc, ed_ref[...], precision=_HIGH,
                preferred_element_type=jnp.float32)
    a_ref[...] = A
    m = jnp.max(A, axis=-1, keepdims=True)
    lse_ref[...] = jnp.log(jnp.sum(jnp.exp(A - m), axis=-1,
                                   keepdims=True)) + m
    kl_ref[...] = jnp.full((1, 1), 0.0) + kl

  return pl.pallas_call(
      body,
      out_shape=[
          jax.ShapeDtypeStruct((N_GRAPHS, N_GRAPHS), jnp.float32),
          jax.ShapeDtypeStruct((N_GRAPHS, N_ATTS), jnp.float32),
          jax.ShapeDtypeStruct((N_GRAPHS, 1), jnp.float32),
          jax.ShapeDtypeStruct((1, 1), jnp.float32),
      ],
  )(g, W_mean, b_mean, W_var, b_var, W_dec, emb_dec)


def _tc_edge_sum(q3):
  def body(q_ref, o_ref):
    i = pl.program_id(0)
    sp = jax.nn.softplus(-q_ref[0])

    @pl.when(i == 0)
    def _():
      o_ref[...] = jnp.zeros_like(o_ref)

    o_ref[...] += jnp.sum(sp, axis=0, keepdims=True)

  return pl.pallas_call(
      body,
      grid=(10,),
      in_specs=[pl.BlockSpec((1, EROWS // 10, 128), lambda i: (i, 0, 0))],
      out_specs=pl.BlockSpec((1, 128), lambda i: (0, 0)),
      out_shape=jax.ShapeDtypeStruct((1, 128), jnp.float32),
  )(q3)


def kernel(edge_index, node_atts, batch, emb, W_msg0, W_self0, b0,
           W_msg1, W_self1, b1, W_mean, b_mean, W_var, b_var,
           W_dec, emb_dec):
  ei3 = edge_index.reshape(2, EROWS, 128)
  atts3 = node_atts.reshape(10, 1, _BLK)
  batch3 = batch.reshape(10, 1, _BLK)
  batch_pad = jnp.concatenate(
      [batch, jnp.zeros((NPAD - N_NODES,), jnp.int32)])
  atts_pad = jnp.concatenate(
      [node_atts, jnp.zeros((NPAD - N_NODES,), jnp.int32)])
  b0_2 = b0.reshape(1, NDIM)
  b1_2 = b1.reshape(1, NDIM)
  bm_2 = b_mean.reshape(1, SDIM)
  bv_2 = b_var.reshape(1, SDIM)

  h0 = _tc_embed(atts3, emb)
  p = _sc_agg(h0, ei3)
  h1 = _tc_layer(p[0], p[1], h0, W_msg0, W_self0, b0_2)
  p2 = _sc_agg(h1, ei3)
  g = _tc_layer_pool(p2[0], p2[1], h1, W_msg1, W_self1, b1_2, batch3)
  Q, A, lse2, kl = _tc_heads(g, W_mean, bm_2, W_var, bv_2, W_dec, emb_dec)
  q, att_parts = _sc_decoder(ei3, batch_pad, atts_pad,
                             Q.reshape(-1), A.reshape(-1), lse2.reshape(-1))
  esum = _tc_edge_sum(q.reshape(10, EROWS // 10, 128))
  recon_edge = jnp.sum(esum)
  recon_att = jnp.sum(att_parts)
  return recon_edge + recon_att + BETA * kl[0, 0]


# speedup vs baseline: 13.8541x; 1.1905x over previous
import dataclasses
import functools

import jax
import jax.numpy as jnp
from jax import lax
from jax.experimental import pallas as pl
from jax.experimental.pallas import tpu as pltpu
from jax.experimental.pallas import tpu_sc as plsc

NDIM = 128
SDIM = 64
N_NODES = 10000
N_EDGES = 320000
N_GRAPHS = 256
N_ATTS = 32
BETA = 0.005

NC = 2
NS = 16
NW = NC * NS
EROWS = N_EDGES // 128
ROWS_PER_TILE = EROWS // NW
EXTRA_ROWS = EROWS - ROWS_PER_TILE * NW
NPAD = 10240

_HIGH = jax.lax.Precision.HIGHEST


def _mesh():
  return plsc.VectorSubcoreMesh(core_axis_name="c", subcore_axis_name="s")


def _sc_params():
  cp = pltpu.CompilerParams()
  if "needs_layout_passes" in pltpu.CompilerParams.__dataclass_fields__:
    cp = dataclasses.replace(cp, needs_layout_passes=False)
  return cp


def _sc_agg(h, ei3):
  @functools.partial(
      pl.kernel,
      out_type=jax.ShapeDtypeStruct((NC, N_NODES, NDIM), jnp.float32),
      mesh=_mesh(),
      compiler_params=_sc_params(),
      scratch_types=[
          pltpu.VMEM((128,), jnp.int32),
          pltpu.VMEM((128,), jnp.int32),
          pltpu.VMEM((128, NDIM), jnp.float32),
          pltpu.VMEM((128,), jnp.int32),
          pltpu.VMEM((128,), jnp.int32),
          pltpu.VMEM((128, NDIM), jnp.float32),
          pltpu.VMEM_SHARED((N_NODES, NDIM), jnp.float32),
          pltpu.SemaphoreType.DMA,
          pltpu.SemaphoreType.DMA,
      ],
  )
  def k(h_hbm, ei_hbm, out_hbm, srcA, dstA, rowsA, srcB, dstB, rowsB,
        agg_sh, semA, semB):
    cid = lax.axis_index("c")
    sid = lax.axis_index("s")
    tid = cid * NS + sid

    z16 = jnp.zeros((16,), jnp.float32)

    @pl.loop(0, 128)
    def _(r):
      for c in range(NDIM // 16):
        rowsA[r, pl.ds(c * 16, 16)] = z16

    @pl.loop(0, 4)
    def _(b):
      pltpu.sync_copy(rowsA, agg_sh.at[pl.ds(sid * 624 + b * 128, 128)])

    pltpu.sync_copy(rowsA.at[pl.ds(0, 112)],
                    agg_sh.at[pl.ds(sid * 624 + 512, 112)])

    @pl.when(sid == 0)
    def _():
      pltpu.sync_copy(rowsA.at[pl.ds(0, 16)], agg_sh.at[pl.ds(9984, 16)])

    plsc.subcore_barrier()

    r0 = tid * ROWS_PER_TILE

    pltpu.sync_copy(ei_hbm.at[0, r0], srcA)
    pltpu.make_async_copy(h_hbm.at[srcA], rowsA, semA).start()

    @pl.loop(0, ROWS_PER_TILE // 2)
    def _(j):
      rA = r0 + 2 * j
      rB = rA + 1
      pltpu.sync_copy(ei_hbm.at[0, rB], srcB)
      pltpu.make_async_copy(h_hbm.at[srcB], rowsB, semB).start()
      pltpu.make_async_copy(h_hbm.at[srcA], rowsA, semA).wait()
      pltpu.sync_copy(ei_hbm.at[1, rA], dstA)
      pltpu.sync_copy(rowsA, agg_sh.at[dstA], add=True)
      @pl.when(j < ROWS_PER_TILE // 2 - 1)
      def _():
        pltpu.sync_copy(ei_hbm.at[0, rA + 2], srcA)
        pltpu.make_async_copy(h_hbm.at[srcA], rowsA, semA).start()
      pltpu.make_async_copy(h_hbm.at[srcB], rowsB, semB).wait()
      pltpu.sync_copy(ei_hbm.at[1, rB], dstB)
      pltpu.sync_copy(rowsB, agg_sh.at[dstB], add=True)

    @pl.when(tid < EXTRA_ROWS)
    def _():
      r = NW * ROWS_PER_TILE + tid
      pltpu.sync_copy(ei_hbm.at[0, r], srcA)
      pltpu.async_copy(h_hbm.at[srcA], rowsA, semA).wait()
      pltpu.sync_copy(ei_hbm.at[1, r], dstA)
      pltpu.sync_copy(rowsA, agg_sh.at[dstA], add=True)

    plsc.subcore_barrier()

    pltpu.sync_copy(agg_sh.at[pl.ds(sid * 624, 624)],
                    out_hbm.at[cid].at[pl.ds(sid * 624, 624)])

    @pl.when(sid == 0)
    def _():
      pltpu.sync_copy(agg_sh.at[pl.ds(9984, 16)],
                      out_hbm.at[cid].at[pl.ds(9984, 16)])

  return k(h, ei3)


def _sc_cin(ei3, atts_pad):
  @functools.partial(
      pl.kernel,
      out_type=jax.ShapeDtypeStruct((NC, N_NODES, 128), jnp.float32),
      mesh=_mesh(),
      compiler_params=_sc_params(),
      scratch_types=[
          pltpu.VMEM((NPAD,), jnp.int32),
          pltpu.VMEM((8, 128), jnp.int32),
          pltpu.VMEM((8, 128), jnp.int32),
          pltpu.VMEM((128, 128), jnp.float32),
          pltpu.VMEM((128, 128), jnp.float32),
          pltpu.VMEM_SHARED((N_NODES, 128), jnp.float32),
          pltpu.SemaphoreType.DMA,
          pltpu.SemaphoreType.DMA,
      ],
  )
  def k(ei_hbm, a_hbm, out_hbm, a_v, src_v, dst_v, rowsA, rowsB, cin_sh,
        semA, semB):
    cid = lax.axis_index("c")
    sid = lax.axis_index("s")
    tid = cid * NS + sid

    pltpu.sync_copy(a_hbm, a_v)

    z16 = jnp.zeros((16,), jnp.float32)

    @pl.loop(0, 128)
    def _(r):
      for c in range(128 // 16):
        rowsA[r, pl.ds(c * 16, 16)] = z16
        rowsB[r, pl.ds(c * 16, 16)] = z16

    @pl.loop(0, 4)
    def _(b):
      pltpu.sync_copy(rowsA, cin_sh.at[pl.ds(sid * 624 + b * 128, 128)])

    pltpu.sync_copy(rowsA.at[pl.ds(0, 112)],
                    cin_sh.at[pl.ds(sid * 624 + 512, 112)])

    @pl.when(sid == 0)
    def _():
      pltpu.sync_copy(rowsA.at[pl.ds(0, 16)], cin_sh.at[pl.ds(9984, 16)])

    plsc.subcore_barrier()

    ones16 = jnp.full((16,), 1.0, jnp.float32)
    bufs = (rowsA, rowsB)
    sems = (semA, semB)

    def scat(buf, i, vals):
      for c in range(8):
        s16 = src_v[i, pl.ds(c * 16, 16)]
        a16 = plsc.load_gather(a_v, [s16])
        e16 = c * 16 + lax.iota(jnp.int32, 16)
        plsc.store_scatter(buf, [e16, a16], vals)

    def do_chunk(row_base, nrows):
      row_base = pl.multiple_of(row_base, 8)
      pltpu.sync_copy(ei_hbm.at[0, pl.ds(row_base, nrows)],
                      src_v.at[pl.ds(0, nrows)])
      pltpu.sync_copy(ei_hbm.at[1, pl.ds(row_base, nrows)],
                      dst_v.at[pl.ds(0, nrows)])

      for i in range(nrows):
        buf, sem = bufs[i % 2], sems[i % 2]
        if i >= 2:
          pltpu.make_async_copy(buf, cin_sh.at[dst_v.at[i - 2]], sem).wait()
          scat(buf, i - 2, z16)
        scat(buf, i, ones16)
        pltpu.async_copy(buf, cin_sh.at[dst_v.at[i]], sem, add=True)

      for i in range(max(0, nrows - 2), nrows):
        buf, sem = bufs[i % 2], sems[i % 2]
        pltpu.make_async_copy(buf, cin_sh.at[dst_v.at[i]], sem).wait()
        scat(buf, i, z16)

    nchunks = jnp.where(tid < 24, 10, 9)

    @pl.loop(0, nchunks)
    def _(j):
      do_chunk(8 * (tid + NW * j), 8)

    @pl.when(tid == NW - 1)
    def _():
      do_chunk(8 * 312, 4)

    plsc.subcore_barrier()

    pltpu.sync_copy(cin_sh.at[pl.ds(sid * 624, 624)],
                    out_hbm.at[cid].at[pl.ds(sid * 624, 624)])

    @pl.when(sid == 0)
    def _():
      pltpu.sync_copy(cin_sh.at[pl.ds(9984, 16)],
                      out_hbm.at[cid].at[pl.ds(9984, 16)])

  return k(ei3, atts_pad)


_CH = 26


def _sc_decoder(ei3, batch_pad, atts_pad, qflat, aflat, lse):
  @functools.partial(
      pl.kernel,
      out_type=[
          jax.ShapeDtypeStruct((EROWS, 128), jnp.float32),
          jax.ShapeDtypeStruct((NW, 16), jnp.float32),
      ],
      mesh=_mesh(),
      compiler_params=_sc_params(),
      scratch_types=[
          pltpu.VMEM((NPAD,), jnp.int32),
          pltpu.VMEM((NPAD,), jnp.int32),
          pltpu.VMEM((N_GRAPHS * N_GRAPHS,), jnp.float32),
          pltpu.VMEM((N_GRAPHS * N_ATTS,), jnp.float32),
          pltpu.VMEM((N_GRAPHS,), jnp.float32),
          pltpu.VMEM((8, 128), jnp.int32),
          pltpu.VMEM((8, 128), jnp.int32),
          pltpu.VMEM((8, 128), jnp.float32),
          pltpu.VMEM((16,), jnp.float32),
      ],
  )
  def k(ei_hbm, b_hbm, a_hbm, q_hbm, af_hbm, lse_hbm, qout_hbm, att_hbm,
        b_v, a_v, Q_v, A_v, l_v, src_v, dst_v, qc_v, acc_v):
    cid = lax.axis_index("c")
    sid = lax.axis_index("s")
    tid = cid * NS + sid

    pltpu.sync_copy(b_hbm, b_v)
    pltpu.sync_copy(a_hbm, a_v)
    pltpu.sync_copy(q_hbm, Q_v)
    pltpu.sync_copy(af_hbm, A_v)
    pltpu.sync_copy(lse_hbm, l_v)

    def do_rows(row_base, nrows):
      row_base = pl.multiple_of(row_base, 8)
      pltpu.sync_copy(ei_hbm.at[0, pl.ds(row_base, nrows)],
                      src_v.at[pl.ds(0, nrows)])
      pltpu.sync_copy(ei_hbm.at[1, pl.ds(row_base, nrows)],
                      dst_v.at[pl.ds(0, nrows)])

      @pl.loop(0, nrows)
      def _(i):
        for c in range(8):
          s16 = src_v[i, pl.ds(c * 16, 16)]
          d16 = dst_v[i, pl.ds(c * 16, 16)]
          bs = plsc.load_gather(b_v, [s16])
          bd = plsc.load_gather(b_v, [d16])
          q16 = plsc.load_gather(Q_v, [bs * N_GRAPHS + bd])
          qc_v[i, pl.ds(c * 16, 16)] = q16

      pltpu.sync_copy(qc_v.at[pl.ds(0, nrows)],
                      qout_hbm.at[pl.ds(row_base, nrows)])

    nchunks = jnp.where(tid < 24, 10, 9)

    @pl.loop(0, nchunks)
    def _(j):
      do_rows(8 * (tid + NW * j), 8)

    @pl.when(tid == NW - 1)
    def _():
      do_rows(8 * 312, 4)

    nbase = tid * (NPAD // NW)

    def att_step(j, acc):
      n0 = pl.multiple_of(nbase + j * 16, 16)
      b16 = b_v[pl.ds(n0, 16)]
      a16 = a_v[pl.ds(n0, 16)]
      l16 = plsc.load_gather(l_v, [b16])
      af16 = plsc.load_gather(A_v, [b16 * N_ATTS + a16])
      nid = n0 + lax.iota(jnp.int32, 16)
      return acc + jnp.where(nid < N_NODES, l16 - af16, 0.0)

    acc = jnp.zeros((16,), jnp.float32)
    acc = pl.loop(0, NPAD // NW // 16, init_carry=acc)(att_step)
    acc_v[...] = acc
    pltpu.sync_copy(acc_v, att_hbm.at[tid])

  return k(ei3, batch_pad, atts_pad, qflat, aflat, lse)


_BLK = 1000


def _tc_layer1(c0, c1, atts3, emb_pad, emb, Wm, Ws, b2):
  def body(c0_ref, c1_ref, a_ref, ep_ref, e_ref, wm_ref, ws_ref, b_ref,
           o_ref):
    M0 = jnp.dot(ep_ref[...], wm_ref[...], precision=_HIGH,
                 preferred_element_type=jnp.float32)
    S0 = jnp.dot(e_ref[...], ws_ref[...], precision=_HIGH,
                 preferred_element_type=jnp.float32)
    cin = c0_ref[...] + c1_ref[...]
    a = a_ref[0, 0, :]
    onehot = (a[:, None] == lax.broadcasted_iota(jnp.int32, (_BLK, N_ATTS), 1)
              ).astype(jnp.float32)
    x = (jnp.dot(cin, M0, precision=_HIGH,
                 preferred_element_type=jnp.float32)
         + jnp.dot(onehot, S0, precision=_HIGH,
                   preferred_element_type=jnp.float32)
         + b_ref[...])
    o_ref[...] = jnp.tanh(x)

  return pl.pallas_call(
      body,
      grid=(N_NODES // _BLK,),
      in_specs=[
          pl.BlockSpec((_BLK, 128), lambda i: (i, 0)),
          pl.BlockSpec((_BLK, 128), lambda i: (i, 0)),
          pl.BlockSpec((1, 1, _BLK), lambda i: (i, 0, 0)),
          pl.BlockSpec((128, NDIM), lambda i: (0, 0)),
          pl.BlockSpec((N_ATTS, NDIM), lambda i: (0, 0)),
          pl.BlockSpec((NDIM, NDIM), lambda i: (0, 0)),
          pl.BlockSpec((NDIM, NDIM), lambda i: (0, 0)),
          pl.BlockSpec((1, NDIM), lambda i: (0, 0)),
      ],
      out_specs=pl.BlockSpec((_BLK, NDIM), lambda i: (i, 0)),
      out_shape=jax.ShapeDtypeStruct((N_NODES, NDIM), jnp.float32),
  )(c0, c1, atts3, emb_pad, emb, Wm, Ws, b2)


def _tc_layer_pool(p0, p1, h, Wm, Ws, b2, batch3):
  def body(p0_ref, p1_ref, h_ref, wm_ref, ws_ref, b_ref, bat_ref, g_ref):
    i = pl.program_id(0)
    agg = p0_ref[...] + p1_ref[...]
    x = (jnp.dot(agg, wm_ref[...], precision=_HIGH,
                 preferred_element_type=jnp.float32)
         + jnp.dot(h_ref[...], ws_ref[...], precision=_HIGH,
                   preferred_element_type=jnp.float32)
         + b_ref[...])
    h2 = jnp.tanh(x)
    bat = bat_ref[0, 0, :]
    onehot = (bat[:, None] == lax.broadcasted_iota(jnp.int32,
                                                   (_BLK, N_GRAPHS), 1)
              ).astype(jnp.float32)
    gpart = lax.dot_general(onehot, h2, (((0,), (0,)), ((), ())),
                            precision=_HIGH,
                            preferred_element_type=jnp.float32)

    @pl.when(i == 0)
    def _():
      g_ref[...] = jnp.zeros_like(g_ref)

    g_ref[...] += gpart

  return pl.pallas_call(
      body,
      grid=(N_NODES // _BLK,),
      in_specs=[
          pl.BlockSpec((_BLK, NDIM), lambda i: (i, 0)),
          pl.BlockSpec((_BLK, NDIM), lambda i: (i, 0)),
          pl.BlockSpec((_BLK, NDIM), lambda i: (i, 0)),
          pl.BlockSpec((NDIM, NDIM), lambda i: (0, 0)),
          pl.BlockSpec((NDIM, NDIM), lambda i: (0, 0)),
          pl.BlockSpec((1, NDIM), lambda i: (0, 0)),
          pl.BlockSpec((1, 1, _BLK), lambda i: (i, 0, 0)),
      ],
      out_specs=pl.BlockSpec((N_GRAPHS, NDIM), lambda i: (0, 0)),
      out_shape=jax.ShapeDtypeStruct((N_GRAPHS, NDIM), jnp.float32),
  )(p0, p1, h, Wm, Ws, b2, batch3)


def _tc_heads(g, W_mean, b_mean, W_var, b_var, W_dec, emb_dec):
  def body(g_ref, wm_ref, bm_ref, wv_ref, bv_ref, wd_ref, ed_ref,
           q_ref, a_ref, lse_ref, kl_ref):
    gg = g_ref[...]
    mean = jnp.dot(gg, wm_ref[...], precision=_HIGH,
                   preferred_element_type=jnp.float32) + bm_ref[...]
    var = jnp.dot(gg, wv_ref[...], precision=_HIGH,
                  preferred_element_type=jnp.float32) + bv_ref[...]
    kl = -0.5 * jnp.sum(1.0 + var - jnp.square(mean) - jnp.exp(var))
    zc = jnp.dot(mean, wd_ref[...], precision=_HIGH,
                 preferred_element_type=jnp.float32)
    q_ref[...] = lax.dot_general(zc, zc, (((1,), (1,)), ((), ())),
                                 precision=_HIGH,
                                 preferred_element_type=jnp.float32)
    A = jnp.dot(zc, ed_ref[...], precision=_HIGH,
                preferred_element_type=jnp.float32)
    a_ref[...] = A
    m = jnp.max(A, axis=-1, keepdims=True)
    lse_ref[...] = jnp.log(jnp.sum(jnp.exp(A - m), axis=-1,
                                   keepdims=True)) + m
    kl_ref[...] = jnp.full((1, 1), 0.0) + kl

  return pl.pallas_call(
      body,
      out_shape=[
          jax.ShapeDtypeStruct((N_GRAPHS, N_GRAPHS), jnp.float32),
          jax.ShapeDtypeStruct((N_GRAPHS, N_ATTS), jnp.float32),
          jax.ShapeDtypeStruct((N_GRAPHS, 1), jnp.float32),
          jax.ShapeDtypeStruct((1, 1), jnp.float32),
      ],
  )(g, W_mean, b_mean, W_var, b_var, W_dec, emb_dec)


def _tc_edge_sum(q3):
  def body(q_ref, o_ref):
    i = pl.program_id(0)
    sp = jax.nn.softplus(-q_ref[0])

    @pl.when(i == 0)
    def _():
      o_ref[...] = jnp.zeros_like(o_ref)

    o_ref[...] += jnp.sum(sp, axis=0, keepdims=True)

  return pl.pallas_call(
      body,
      grid=(10,),
      in_specs=[pl.BlockSpec((1, EROWS // 10, 128), lambda i: (i, 0, 0))],
      out_specs=pl.BlockSpec((1, 128), lambda i: (0, 0)),
      out_shape=jax.ShapeDtypeStruct((1, 128), jnp.float32),
  )(q3)


def kernel(edge_index, node_atts, batch, emb, W_msg0, W_self0, b0,
           W_msg1, W_self1, b1, W_mean, b_mean, W_var, b_var,
           W_dec, emb_dec):
  ei3 = edge_index.reshape(2, EROWS, 128)
  atts3 = node_atts.reshape(10, 1, _BLK)
  batch3 = batch.reshape(10, 1, _BLK)
  batch_pad = jnp.concatenate(
      [batch, jnp.zeros((NPAD - N_NODES,), jnp.int32)])
  atts_pad = jnp.concatenate(
      [node_atts, jnp.zeros((NPAD - N_NODES,), jnp.int32)])
  b0_2 = b0.reshape(1, NDIM)
  b1_2 = b1.reshape(1, NDIM)
  bm_2 = b_mean.reshape(1, SDIM)
  bv_2 = b_var.reshape(1, SDIM)

  emb_pad = jnp.concatenate([emb, jnp.zeros((128 - N_ATTS, NDIM),
                                             jnp.float32)])
  cp = _sc_cin(ei3, atts_pad)
  h1 = _tc_layer1(cp[0], cp[1], atts3, emb_pad, emb, W_msg0, W_self0, b0_2)
  p2 = _sc_agg(h1, ei3)
  g = _tc_layer_pool(p2[0], p2[1], h1, W_msg1, W_self1, b1_2, batch3)
  Q, A, lse2, kl = _tc_heads(g, W_mean, bm_2, W_var, bv_2, W_dec, emb_dec)
  q, att_parts = _sc_decoder(ei3, batch_pad, atts_pad,
                             Q.reshape(-1), A.reshape(-1), lse2.reshape(-1))
  esum = _tc_edge_sum(q.reshape(10, EROWS // 10, 128))
  recon_edge = jnp.sum(esum)
  recon_att = jnp.sum(att_parts)
  return recon_edge + recon_att + BETA * kl[0, 0]
